# baseline (speedup 1.0000x reference)
import functools

import jax
import jax.numpy as jnp
from jax import lax
from jax.experimental import pallas as pl
from jax.experimental.pallas import tpu as pltpu
from jax.experimental.pallas import tpu_sc as plsc

VOCAB = 1000000
D = 64
B = 16384
NC, NS, L = 2, 16, 16
NW = NC * NS
BPW = B // NW
CHUNK = 128
NCHUNK = BPW // CHUNK
MAGIC = 12582912.0


def _sc_embed(x, weights, scales):
    mesh = plsc.VectorSubcoreMesh(core_axis_name="c", subcore_axis_name="s")

    @functools.partial(
        pl.kernel,
        mesh=mesh,
        out_type=jax.ShapeDtypeStruct((B, D), jnp.float32),
        compiler_params=pltpu.CompilerParams(use_tc_tiling_on_sc=False),
        scratch_types=[
            pltpu.VMEM((BPW,), jnp.int32),
            pltpu.VMEM((BPW, D), jnp.float32),
            pltpu.VMEM((BPW,), jnp.float32),
            pltpu.SemaphoreType.DMA,
        ],
    )
    def k(x_hbm, w_hbm, s_hbm, out_hbm, idx_v, rows_v, sc_v, sem):
        wid = lax.axis_index("s") * NC + lax.axis_index("c")
        base = wid * BPW
        pltpu.sync_copy(x_hbm.at[pl.ds(base, BPW)], idx_v)
        copies = []
        for j in range(NCHUNK):
            sl = pl.ds(j * CHUNK, CHUNK)
            copies.append(pltpu.async_copy(w_hbm.at[idx_v.at[sl]], rows_v.at[sl], sem))
            copies.append(pltpu.async_copy(s_hbm.at[idx_v.at[sl]], sc_v.at[sl], sem))
        for cp in copies:
            cp.wait()

        def grp_body(g, carry):
            r0 = g * L
            scg = sc_v[pl.ds(r0, L)]
            for i in range(L):
                sc = scg[i]
                for c in range(D // L):
                    v = rows_v[r0 + i, pl.ds(c * L, L)]
                    v = (v + MAGIC) - MAGIC
                    v = jnp.minimum(jnp.maximum(v, -128.0), 127.0)
                    rows_v[r0 + i, pl.ds(c * L, L)] = v * sc
            return carry

        lax.fori_loop(0, BPW // L, grp_body, 0)
        pltpu.sync_copy(rows_v, out_hbm.at[pl.ds(base, BPW)])

    return k(x, weights, scales)


def kernel(x, weights, scales):
    return _sc_embed(x.astype(jnp.int32), weights, scales)
